# Initial kernel scaffold; baseline (speedup 1.0000x reference)
import jax
import jax.numpy as jnp
from jax.experimental import pallas as pl


def kernel(x, edge_index, W, b, gamma, beta):
    raise NotImplementedError("write your pallas kernel here")


# speedup vs baseline: 7.6846x; 7.6846x over previous
import functools

import jax
import jax.numpy as jnp
from jax import lax
from jax.experimental import pallas as pl
from jax.experimental.pallas import tpu as pltpu
from jax.experimental.pallas import tpu_sc as plsc

N = 10000
E = 320000
D = 128

NC = 2
NS = 16
EB = 125
NBT = E // EB
NBW = NBT // (NC * NS)
ROWS_PER_SUB = 624
TAIL_START = ROWS_PER_SUB * NS
TAIL_ROWS = N - TAIL_START


def _relu_body(x_ref, h_ref):
    h_ref[...] = jnp.maximum(x_ref[...], 0.0)


def _final_body(h_ref, p_ref, w_ref, b_ref, g_ref, be_ref, o_ref):
    pre = h_ref[...] + p_ref[0] + p_ref[1]
    out = jnp.dot(pre, w_ref[...].T, preferred_element_type=jnp.float32)
    out = out + b_ref[...]
    mean = jnp.mean(out, axis=0, keepdims=True)
    var = jnp.mean((out - mean) ** 2, axis=0, keepdims=True)
    o_ref[...] = (out - mean) * lax.rsqrt(var + 1e-5) * g_ref[...] + be_ref[...]


def _sc_segment_sum(h, src3, dst3, zeros):
    mesh = plsc.VectorSubcoreMesh(core_axis_name="c", subcore_axis_name="s")

    @functools.partial(
        pl.kernel,
        out_type=jax.ShapeDtypeStruct((NC, N, D), jnp.float32),
        mesh=mesh,
        scratch_types=[
            pltpu.VMEM((NBW, 1, EB), jnp.int32),
            pltpu.VMEM((NBW, 1, EB), jnp.int32),
            pltpu.VMEM((EB, D), jnp.float32),
            pltpu.VMEM_SHARED((N, D), jnp.float32),
            pltpu.SemaphoreType.DMA,
        ],
    )
    def seg_sum(h_hbm, src_hbm, dst_hbm, zeros_hbm, out_hbm,
                src_v, dst_v, rows_v, agg_sh, gsem):
        c = lax.axis_index("c")
        s = lax.axis_index("s")
        pltpu.sync_copy(zeros_hbm.at[pl.ds(s * ROWS_PER_SUB, ROWS_PER_SUB)],
                        agg_sh.at[pl.ds(s * ROWS_PER_SUB, ROWS_PER_SUB)])

        @pl.when(s == NS - 1)
        def _zero_tail():
            pltpu.sync_copy(zeros_hbm.at[pl.ds(TAIL_START, TAIL_ROWS)],
                            agg_sh.at[pl.ds(TAIL_START, TAIL_ROWS)])
        b0 = (c * NS + s) * NBW
        pltpu.sync_copy(src_hbm.at[pl.ds(b0, NBW)], src_v)
        pltpu.sync_copy(dst_hbm.at[pl.ds(b0, NBW)], dst_v)
        plsc.subcore_barrier()

        def body(j, carry):
            pltpu.async_copy(h_hbm.at[src_v.at[j, 0]], rows_v, gsem).wait()
            pltpu.sync_copy(rows_v, agg_sh.at[dst_v.at[j, 0]], add=True)
            return carry

        lax.fori_loop(0, NBW, body, 0, unroll=False)

        plsc.subcore_barrier()
        pltpu.sync_copy(agg_sh.at[pl.ds(s * ROWS_PER_SUB, ROWS_PER_SUB)],
                        out_hbm.at[c, pl.ds(s * ROWS_PER_SUB, ROWS_PER_SUB)])

        @pl.when(s == NS - 1)
        def _flush_tail():
            pltpu.sync_copy(agg_sh.at[pl.ds(TAIL_START, TAIL_ROWS)],
                            out_hbm.at[c, pl.ds(TAIL_START, TAIL_ROWS)])

    return seg_sum(h, src3, dst3, zeros)


def kernel(x, edge_index, W, b, gamma, beta):
    h = pl.pallas_call(
        _relu_body,
        out_shape=jax.ShapeDtypeStruct((N, D), jnp.float32),
    )(x)

    src3 = edge_index[0].reshape(NBT, 1, EB)
    dst3 = edge_index[1].reshape(NBT, 1, EB)
    zeros = jnp.zeros((N, D), jnp.float32)
    parts = _sc_segment_sum(h, src3, dst3, zeros)

    out = pl.pallas_call(
        _final_body,
        out_shape=jax.ShapeDtypeStruct((N, D), jnp.float32),
    )(h, parts, W, b.reshape(1, D), gamma.reshape(1, D), beta.reshape(1, D))
    return out


# speedup vs baseline: 12.1463x; 1.5806x over previous
import functools

import jax
import jax.numpy as jnp
from jax import lax
from jax.experimental import pallas as pl
from jax.experimental.pallas import tpu as pltpu
from jax.experimental.pallas import tpu_sc as plsc

N = 10000
E = 320000
D = 128

NC = 2
NS = 16
EB = 125
NBT = E // EB
NBW = NBT // (NC * NS)
ROWS_PER_SUB = 624
TAIL_START = ROWS_PER_SUB * NS
TAIL_ROWS = N - TAIL_START
NBUF = 3
NI = 2 * NBUF
GRP = NI
NGRP = NBW // GRP
TAILJ = NGRP * GRP


def _relu_body(x_ref, h_ref):
    h_ref[...] = jnp.maximum(x_ref[...], 0.0)


def _final_body(h_ref, p_ref, w_ref, b_ref, g_ref, be_ref, o_ref):
    pre = h_ref[...] + p_ref[0] + p_ref[1]
    out = jnp.dot(pre, w_ref[...].T, preferred_element_type=jnp.float32)
    out = out + b_ref[...]
    mean = jnp.mean(out, axis=0, keepdims=True)
    var = jnp.mean((out - mean) ** 2, axis=0, keepdims=True)
    o_ref[...] = (out - mean) * lax.rsqrt(var + 1e-5) * g_ref[...] + be_ref[...]


def _sc_segment_sum(h, eidx3, zeros):
    mesh = plsc.VectorSubcoreMesh(core_axis_name="c", subcore_axis_name="s")

    @functools.partial(
        pl.kernel,
        out_type=jax.ShapeDtypeStruct((NC, N, D), jnp.float32),
        mesh=mesh,
        scratch_types=[
            pltpu.VMEM((NI, 2, EB), jnp.int32),
            pltpu.VMEM((NBUF, EB, D), jnp.float32),
            pltpu.VMEM_SHARED((N, D), jnp.float32),
            [pltpu.SemaphoreType.DMA] * NI,
            [pltpu.SemaphoreType.DMA] * NBUF,
        ],
    )
    def seg_sum(h_hbm, eidx_hbm, zeros_hbm, out_hbm,
                idx_v, rows_v, agg_sh, isems, gsems):
        c = lax.axis_index("c")
        s = lax.axis_index("s")
        pltpu.sync_copy(zeros_hbm.at[pl.ds(s * ROWS_PER_SUB, ROWS_PER_SUB)],
                        agg_sh.at[pl.ds(s * ROWS_PER_SUB, ROWS_PER_SUB)])

        @pl.when(s == NS - 1)
        def _zero_tail():
            pltpu.sync_copy(zeros_hbm.at[pl.ds(TAIL_START, TAIL_ROWS)],
                            agg_sh.at[pl.ds(TAIL_START, TAIL_ROWS)])
        plsc.subcore_barrier()

        b0 = (c * NS + s) * NBW

        def start_idx(j, islot):
            return pltpu.async_copy(eidx_hbm.at[b0 + j], idx_v.at[islot],
                                    isems[islot])

        def start_gather(islot, buf):
            return pltpu.async_copy(h_hbm.at[idx_v.at[islot, 0]],
                                    rows_v.at[buf], gsems[buf])

        def wait_idx(j, islot):
            pltpu.make_async_copy(eidx_hbm.at[b0 + j], idx_v.at[islot],
                                  isems[islot]).wait()

        def wait_gather(islot, buf):
            pltpu.make_async_copy(h_hbm.at[idx_v.at[islot, 0]],
                                  rows_v.at[buf], gsems[buf]).wait()

        def scatter(islot, buf):
            pltpu.sync_copy(rows_v.at[buf], agg_sh.at[idx_v.at[islot, 1]],
                            add=True)

        for k in range(NI):
            start_idx(k, k)
        for k in range(NBUF):
            wait_idx(k, k)
            start_gather(k, k)

        def body(g, carry):
            j0 = g * GRP
            for k in range(GRP):
                j = j0 + k
                buf = k % NBUF
                wait_gather(k, buf)
                scatter(k, buf)

                @pl.when(j + NI < NBW)
                def _prefetch_idx():
                    start_idx(j + NI, k)

                @pl.when(j + NBUF < NBW)
                def _refill():
                    kn = (k + NBUF) % NI
                    wait_idx(j + NBUF, kn)
                    start_gather(kn, buf)
            return carry

        lax.fori_loop(0, NGRP, body, 0, unroll=False)

        for j in range(TAILJ, NBW):
            k = j % NI
            buf = j % NBUF
            wait_gather(k, buf)
            scatter(k, buf)

        plsc.subcore_barrier()
        pltpu.sync_copy(agg_sh.at[pl.ds(s * ROWS_PER_SUB, ROWS_PER_SUB)],
                        out_hbm.at[c, pl.ds(s * ROWS_PER_SUB, ROWS_PER_SUB)])

        @pl.when(s == NS - 1)
        def _flush_tail():
            pltpu.sync_copy(agg_sh.at[pl.ds(TAIL_START, TAIL_ROWS)],
                            out_hbm.at[c, pl.ds(TAIL_START, TAIL_ROWS)])

    return seg_sum(h, eidx3, zeros)


def kernel(x, edge_index, W, b, gamma, beta):
    h = pl.pallas_call(
        _relu_body,
        out_shape=jax.ShapeDtypeStruct((N, D), jnp.float32),
    )(x)

    eidx3 = edge_index.reshape(2, NBT, EB).transpose(1, 0, 2)
    zeros = jnp.zeros((N, D), jnp.float32)
    parts = _sc_segment_sum(h, eidx3, zeros)

    out = pl.pallas_call(
        _final_body,
        out_shape=jax.ShapeDtypeStruct((N, D), jnp.float32),
    )(h, parts, W, b.reshape(1, D), gamma.reshape(1, D), beta.reshape(1, D))
    return out
